# baseline (speedup 1.0000x reference)
import dataclasses
import functools

import jax
import jax.numpy as jnp
from jax import lax
from jax.experimental import pallas as pl
from jax.experimental.pallas import tpu as pltpu
from jax.experimental.pallas import tpu_sc as plsc

N = 10000
E = 320000
NP = 10240
NC = 2
NS = 16
NW = NC * NS
EPW = E // NW
L = 16
NSL = NP // NS

_mesh = plsc.VectorSubcoreMesh(core_axis_name="c", subcore_axis_name="s")

_sc_params = pltpu.CompilerParams()
if "needs_layout_passes" in pltpu.CompilerParams.__dataclass_fields__:
    _sc_params = dataclasses.replace(_sc_params, needs_layout_passes=False)


def _fast_rsqrt(d):
    u = plsc.bitcast(d, jnp.int32)
    x = plsc.bitcast(jnp.int32(0x5F3759DF) - (u >> 1), jnp.float32)
    hd = 0.5 * d
    for _ in range(3):
        x = x * (1.5 - hd * x * x)
    return jnp.where(d > 0, x, 0.0)


def _edge_pass_deg(row_v, col_v, acc4):
    ones = jnp.ones((L,), jnp.float32)
    three = jnp.full((L,), 3, jnp.int32)

    @pl.loop(0, EPW, step=L * 5)
    def _(i):
        for u in range(5):
            r = row_v[pl.ds(i + u * L, L)]
            c = col_v[pl.ds(i + u * L, L)]
            plsc.addupdate_scatter(acc4, [three, r], ones, mask=r != c)


def _edge_pass_prop(row_v, col_v, z_v, acc4):
    @pl.loop(0, EPW, step=L * 5)
    def _(i):
        for u in range(5):
            r = row_v[pl.ds(i + u * L, L)]
            c = col_v[pl.ds(i + u * L, L)]
            m = r != c
            for f in range(3):
                fs = jnp.full((L,), f, jnp.int32)
                v = plsc.load_gather(z_v, [fs, r])
                plsc.addupdate_scatter(acc4, [fs, c], v, mask=m)


def _reduce_acc_and_store(c, s, lo, acc4, accst_hbm, rbuf, sacc, out_ref):
    pltpu.sync_copy(acc4, accst_hbm.at[c, s])
    plsc.subcore_barrier()
    for k in range(4):
        pltpu.sync_copy(
            accst_hbm.at[c, pl.ds(4 * k, 4), :, pl.ds(lo, NSL)], rbuf)

        @pl.loop(0, NSL, step=L)
        def _(j):
            for f in range(3):
                acc = rbuf[0, f, pl.ds(j, L)] if k == 0 \
                    else sacc[f, pl.ds(j, L)] + rbuf[0, f, pl.ds(j, L)]
                for t in range(1, 4):
                    acc = acc + rbuf[t, f, pl.ds(j, L)]
                sacc[f, pl.ds(j, L)] = acc

    pltpu.sync_copy(sacc, out_ref)


@functools.partial(
    pl.kernel,
    out_type=(jax.ShapeDtypeStruct((NC, 4, NP), jnp.float32),
              jax.ShapeDtypeStruct((NC, NP), jnp.float32),
              jax.ShapeDtypeStruct((NC, 3, NP), jnp.float32),
              jax.ShapeDtypeStruct((NC, NS, NP), jnp.float32),
              jax.ShapeDtypeStruct((NC, NS, 4, NP), jnp.float32)),
    mesh=_mesh,
    compiler_params=_sc_params,
    scratch_types=[
        pltpu.VMEM((EPW,), jnp.int32),
        pltpu.VMEM((EPW,), jnp.int32),
        pltpu.VMEM((4, NP), jnp.float32),
        pltpu.VMEM((3, NP), jnp.float32),
        pltpu.VMEM((NSL,), jnp.float32),
        pltpu.VMEM((3, NSL), jnp.float32),
        pltpu.VMEM((3, NSL), jnp.float32),
        pltpu.VMEM((4, 4, NSL), jnp.float32),
        pltpu.VMEM((4, NSL), jnp.float32),
        pltpu.SemaphoreType.DMA,
    ],
)
def _sc_layer1(row_hbm, col_hbm, y1_hbm, zeros_hbm,
               s1_hbm, dinv_hbm, zst_hbm, degst_hbm, accst_hbm,
               row_v, col_v, acc4, z_v, dacc, ybuf, zbuf, rbuf, sacc, sem):
    c = lax.axis_index("c")
    s = lax.axis_index("s")
    lo = s * NSL
    base_a = (2 * s + c) * EPW
    base_b = (2 * s + (1 - c)) * EPW

    c1 = pltpu.async_copy(row_hbm.at[pl.ds(base_b, EPW)], row_v, sem)
    c2 = pltpu.async_copy(col_hbm.at[pl.ds(base_b, EPW)], col_v, sem)
    c3 = pltpu.async_copy(zeros_hbm, acc4, sem)
    c1.wait()
    c2.wait()
    c3.wait()
    _edge_pass_deg(row_v, col_v, acc4)
    c1 = pltpu.async_copy(row_hbm.at[pl.ds(base_a, EPW)], row_v, sem)
    c2 = pltpu.async_copy(col_hbm.at[pl.ds(base_a, EPW)], col_v, sem)
    c1.wait()
    c2.wait()
    _edge_pass_deg(row_v, col_v, acc4)

    pltpu.sync_copy(acc4.at[3], degst_hbm.at[c, s])
    plsc.subcore_barrier()
    for k in range(4):
        pltpu.sync_copy(
            degst_hbm.at[c, pl.ds(4 * k, 4), pl.ds(lo, NSL)], rbuf.at[0])

        @pl.loop(0, NSL, step=L)
        def _(j):
            d = rbuf[0, 0, pl.ds(j, L)] if k == 0 \
                else dacc[pl.ds(j, L)] + rbuf[0, 0, pl.ds(j, L)]
            for t in range(1, 4):
                d = d + rbuf[0, t, pl.ds(j, L)]
            dacc[pl.ds(j, L)] = d

    @pl.loop(0, NSL, step=L)
    def _(j):
        dacc[pl.ds(j, L)] = _fast_rsqrt(dacc[pl.ds(j, L)])

    pltpu.sync_copy(dacc, dinv_hbm.at[c, pl.ds(lo, NSL)])
    pltpu.sync_copy(y1_hbm.at[:, pl.ds(lo, NSL)], ybuf)

    @pl.loop(0, NSL, step=L)
    def _(j):
        dv = dacc[pl.ds(j, L)]
        for f in range(3):
            zbuf[f, pl.ds(j, L)] = dv * ybuf[f, pl.ds(j, L)]

    pltpu.sync_copy(zbuf, zst_hbm.at[c, :, pl.ds(lo, NSL)])
    plsc.subcore_barrier()
    pltpu.sync_copy(zst_hbm.at[c], z_v)

    _edge_pass_prop(row_v, col_v, z_v, acc4)
    _reduce_acc_and_store(c, s, lo, acc4, accst_hbm, rbuf, sacc,
                          s1_hbm.at[c, :, pl.ds(lo, NSL)])


@functools.partial(
    pl.kernel,
    out_type=(jax.ShapeDtypeStruct((NC, 4, NP), jnp.float32),
              jax.ShapeDtypeStruct((NC, 3, NP), jnp.float32),
              jax.ShapeDtypeStruct((NC, 3, NP), jnp.float32),
              jax.ShapeDtypeStruct((NC, NS, 4, NP), jnp.float32)),
    mesh=_mesh,
    compiler_params=_sc_params,
    scratch_types=[
        pltpu.VMEM((EPW,), jnp.int32),
        pltpu.VMEM((EPW,), jnp.int32),
        pltpu.VMEM((4, NP), jnp.float32),
        pltpu.VMEM((3, NP), jnp.float32),
        pltpu.VMEM((NSL,), jnp.float32),
        pltpu.VMEM((3, NSL), jnp.float32),
        pltpu.VMEM((3, NSL), jnp.float32),
        pltpu.VMEM((4, 4, NSL), jnp.float32),
        pltpu.VMEM((4, NSL), jnp.float32),
        pltpu.SemaphoreType.DMA,
    ],
)
def _sc_layer2(row_hbm, col_hbm, s1_hbm, y0_hbm, dinv_hbm, zeros_hbm,
               s2_hbm, h1_hbm, zst_hbm, accst_hbm,
               row_v, col_v, acc4, z_v, dvbuf, ybuf, hbuf, rbuf, sacc, sem):
    c = lax.axis_index("c")
    s = lax.axis_index("s")
    wid = c * NS + s
    lo = s * NSL
    base = wid * EPW

    c1 = pltpu.async_copy(row_hbm.at[pl.ds(base, EPW)], row_v, sem)
    c2 = pltpu.async_copy(col_hbm.at[pl.ds(base, EPW)], col_v, sem)
    c3 = pltpu.async_copy(zeros_hbm, acc4, sem)
    c4 = pltpu.async_copy(s1_hbm.at[0, :, pl.ds(lo, NSL)], rbuf.at[0], sem)
    c5 = pltpu.async_copy(s1_hbm.at[1, :, pl.ds(lo, NSL)], rbuf.at[1], sem)
    c6 = pltpu.async_copy(dinv_hbm.at[c, pl.ds(lo, NSL)], dvbuf, sem)
    c7 = pltpu.async_copy(y0_hbm.at[:, pl.ds(lo, NSL)], ybuf, sem)
    for cc in (c1, c2, c3, c4, c5, c6, c7):
        cc.wait()

    @pl.loop(0, NSL, step=L)
    def _(j):
        dv = dvbuf[pl.ds(j, L)]
        for f in range(3):
            s1 = rbuf[0, f, pl.ds(j, L)] + rbuf[1, f, pl.ds(j, L)]
            h1 = jnp.maximum(ybuf[f, pl.ds(j, L)] - dv * s1, 0.0)
            hbuf[f, pl.ds(j, L)] = h1
            ybuf[f, pl.ds(j, L)] = dv * h1

    pltpu.sync_copy(hbuf, h1_hbm.at[c, :, pl.ds(lo, NSL)])
    pltpu.sync_copy(ybuf, zst_hbm.at[c, :, pl.ds(lo, NSL)])
    plsc.subcore_barrier()
    pltpu.sync_copy(zst_hbm.at[c], z_v)

    _edge_pass_prop(row_v, col_v, z_v, acc4)
    _reduce_acc_and_store(c, s, lo, acc4, accst_hbm, rbuf, sacc,
                          s2_hbm.at[c, :, pl.ds(lo, NSL)])


def _tc_mm1(xp, Wcat, b1c):
    def body(x_ref, w_ref, b1_ref, y0_ref, y1_ref):
        yT = lax.dot_general(w_ref[...], x_ref[...], (((0,), (1,)), ((), ())),
                             preferred_element_type=jnp.float32)
        y0_ref[...] = yT[0:3, :] + b1_ref[...]
        y1_ref[...] = yT[3:6, :]

    return pl.pallas_call(
        body,
        out_shape=(jax.ShapeDtypeStruct((3, NP), jnp.float32),
                   jax.ShapeDtypeStruct((3, NP), jnp.float32)))(xp, Wcat, b1c)


def _tc_final(s2p, h1p, dinvp, W2t0, W2t1, b2c, Wl, blr):
    def body(s2p_ref, h1_ref, dinv_ref, w20_ref, w21_ref, b2_ref, wl_ref,
             bl_ref, o_ref):
        s2 = s2p_ref[0, 0:3, :] + s2p_ref[1, 0:3, :]
        p2 = -dinv_ref[0:1, :] * s2
        h1 = h1_ref[0]
        h2 = lax.dot_general(w20_ref[...], h1, (((1,), (0,)), ((), ())),
                             preferred_element_type=jnp.float32)
        h2 = h2 + lax.dot_general(w21_ref[...], p2, (((1,), (0,)), ((), ())),
                                  preferred_element_type=jnp.float32)
        h2 = jnp.maximum(h2 + b2_ref[...], 0.0)
        out = lax.dot_general(h2, wl_ref[...], (((0,), (0,)), ((), ())),
                              preferred_element_type=jnp.float32)
        o_ref[...] = out + bl_ref[...]

    return pl.pallas_call(
        body, out_shape=jax.ShapeDtypeStruct((NP, 128), jnp.float32))(
            s2p, h1p, dinvp, W2t0, W2t1, b2c, Wl, blr)


def kernel(x, edge_index, W1_0, W1_1, b1, W2_0, W2_1, b2, Wl, bl):
    xp = jnp.pad(x, ((0, NP - N), (0, 0)))
    row = edge_index[0]
    col = edge_index[1]
    Wcat = jnp.concatenate([W1_0, W1_1], axis=1)
    b1c = b1.reshape(3, 1)
    W2t0 = W2_0.T
    W2t1 = W2_1.T
    b2c = b2.reshape(128, 1)
    blr = bl.reshape(1, 128)
    zeros4 = jnp.zeros((4, NP), jnp.float32)

    y0T, y1T = _tc_mm1(xp, Wcat, b1c)
    s1p, dinvp, _, _, _ = _sc_layer1(row, col, y1T, zeros4)
    s2p, h1p, _, _ = _sc_layer2(row, col, s1p, y0T, dinvp, zeros4)
    out = _tc_final(s2p, h1p, dinvp, W2t0, W2t1, b2c, Wl, blr)
    return out[:N]
